# baseline (speedup 1.0000x reference)
import functools
import jax
import jax.numpy as jnp
from jax.experimental import pallas as pl
from jax.experimental.pallas import tpu as pltpu

_B = 16


def _fps_body(xyz_ref, new_ref, *, npoint, n):
    x = xyz_ref[:, 0, :]
    y = xyz_ref[:, 1, :]
    z = xyz_ref[:, 2, :]
    lane_n = jax.lax.broadcasted_iota(jnp.int32, (_B, n), 1)
    lane_s = jax.lax.broadcasted_iota(jnp.int32, (_B, npoint), 1)

    def body(i, state):
        dist, far, ax, ay, az = state
        oh = (lane_n == far).astype(jnp.float32)
        cx = jnp.sum(oh * x, axis=1, keepdims=True)
        cy = jnp.sum(oh * y, axis=1, keepdims=True)
        cz = jnp.sum(oh * z, axis=1, keepdims=True)
        sel = lane_s == i
        ax = jnp.where(sel, cx, ax)
        ay = jnp.where(sel, cy, ay)
        az = jnp.where(sel, cz, az)
        dx = x - cx
        dy = y - cy
        dz = z - cz
        d = dx * dx + dy * dy + dz * dz
        dist = jnp.minimum(dist, d)
        m = jnp.max(dist, axis=1, keepdims=True)
        far = jnp.min(jnp.where(dist == m, lane_n, n), axis=1, keepdims=True)
        return (dist, far, ax, ay, az)

    init = (
        jnp.full((_B, n), 1e10, jnp.float32),
        jnp.zeros((_B, 1), jnp.int32),
        jnp.zeros((_B, npoint), jnp.float32),
        jnp.zeros((_B, npoint), jnp.float32),
        jnp.zeros((_B, npoint), jnp.float32),
    )
    _, _, ax, ay, az = jax.lax.fori_loop(0, npoint, body, init)
    new_ref[:, 0, :] = ax
    new_ref[:, 1, :] = ay
    new_ref[:, 2, :] = az


def _fps(xyz, npoint):
    b, _, n = xyz.shape
    return pl.pallas_call(
        functools.partial(_fps_body, npoint=npoint, n=n),
        out_shape=jax.ShapeDtypeStruct((b, 3, npoint), jnp.float32),
    )(xyz)


def _group_body(new_ref, xyz_ref, pts_ref, out_ref, *, s_, sc, k_, n, r2, cp):
    j = pl.program_id(1)
    nx = new_ref[0]
    px = xyz_ref[0]
    pts = None if pts_ref is None else pts_ref[0]
    nxT = jnp.transpose(nx)
    ns_col = (nxT[:, 0:1] * nxT[:, 0:1] + nxT[:, 1:2] * nxT[:, 1:2]) \
        + nxT[:, 2:3] * nxT[:, 2:3]
    nd_row = (px[0:1, :] * px[0:1, :] + px[1:2, :] * px[1:2, :]) \
        + px[2:3, :] * px[2:3, :]
    dot = jax.lax.dot_general(nx, px, (((0,), (0,)), ((), ())),
                              precision=jax.lax.Precision.DEFAULT,
                              preferred_element_type=jnp.float32)
    d = (-2.0 * dot + ns_col) + nd_row
    mask_f = (d <= r2).astype(jnp.float32)
    lane = jax.lax.broadcasted_iota(jnp.int32, (sc, n), 1)
    a = mask_f
    sh = 1
    while sh < n:
        rolled = pltpu.roll(a, sh, 1)
        a = a + jnp.where(lane >= sh, rolled, 0.0)
        sh *= 2
    ranksel = jnp.where(mask_f > 0.5, jnp.minimum(a - mask_f, 200.0), 222.0)
    rank_t = jnp.transpose(ranksel.astype(jnp.bfloat16))
    count_row = jnp.transpose(
        jnp.sum(mask_f, axis=1, keepdims=True))
    stacked = px if pts is None else jnp.concatenate([px, pts], axis=0)
    hi = stacked.astype(jnp.bfloat16)
    r1 = stacked - hi.astype(jnp.float32)
    mid = r1.astype(jnp.bfloat16)
    lo = (r1 - mid.astype(jnp.float32)).astype(jnp.bfloat16)
    g0 = None
    for k in range(k_):
        sel_t = (rank_t == jnp.bfloat16(k)).astype(jnp.bfloat16)
        dims = (((1,), (0,)), ((), ()))
        g = (jax.lax.dot_general(hi, sel_t, dims,
                                 precision=jax.lax.Precision.DEFAULT,
                                 preferred_element_type=jnp.float32)
             + jax.lax.dot_general(mid, sel_t, dims,
                                   precision=jax.lax.Precision.DEFAULT,
                                   preferred_element_type=jnp.float32)) \
            + jax.lax.dot_general(lo, sel_t, dims,
                                  precision=jax.lax.Precision.DEFAULT,
                                  preferred_element_type=jnp.float32)
        if k == 0:
            empty = (count_row <= 0).astype(jnp.float32)
            g = g + empty * stacked[:, n - 1:n]
            g0 = g
        else:
            pad = (count_row <= k).astype(jnp.float32)
            g = g + pad * g0
        norm = g[0:3] - nx
        rest = g[0:3] if pts is None else g[3:]
        cat = jnp.concatenate([norm, rest], axis=0)
        out_ref[0, :, pl.ds(k * s_ + j * sc, sc)] = cat


def _group(new_xyz, xyz, pts, s_, k_, r2):
    b = xyz.shape[0]
    n = xyz.shape[2]
    cp = 3 if pts is None else pts.shape[1]
    sc = min(s_, 128)
    in_specs = [
        pl.BlockSpec((1, 3, sc), lambda i, j: (i, 0, j)),
        pl.BlockSpec((1, 3, n), lambda i, j: (i, 0, 0)),
    ]
    args = [new_xyz, xyz]
    if pts is None:
        body = functools.partial(
            lambda nr, xr, orf, **kw: _group_body(nr, xr, None, orf, **kw),
            s_=s_, sc=sc, k_=k_, n=n, r2=r2, cp=cp)
    else:
        body = functools.partial(_group_body, s_=s_, sc=sc, k_=k_, n=n,
                                 r2=r2, cp=cp)
        in_specs.append(pl.BlockSpec((1, cp, n), lambda i, j: (i, 0, 0)))
        args.append(pts)
    return pl.pallas_call(
        body,
        grid=(b, s_ // sc),
        in_specs=in_specs,
        out_specs=pl.BlockSpec((1, 3 + cp, k_ * s_), lambda i, j: (i, 0, 0)),
        out_shape=jax.ShapeDtypeStruct((b, 3 + cp, k_ * s_), jnp.float32),
    )(*args)


def _layer_body(x_ref, w_ref, b_ref, y_ref, st_ref, *, nb, count_prev):
    bi = pl.program_id(0)
    x = x_ref[0]
    y = jax.lax.dot_general(w_ref[...], x, (((1,), (0,)), ((), ())),
                            precision=jax.lax.Precision.DEFAULT,
                            preferred_element_type=jnp.float32) + b_ref[...]
    ps = jnp.sum(y, axis=1, keepdims=True)
    pq = jnp.sum(y * y, axis=1, keepdims=True)
    part = jnp.concatenate([ps, pq], axis=1)

    @pl.when(bi == 0)
    def _():
        st_ref[...] = part

    @pl.when(bi > 0)
    def _():
        st_ref[...] = st_ref[...] + part

    y_ref[0] = y


def _layer_norm_body(x_ref, sin_ref, w_ref, b_ref, y_ref, st_ref, *, nb,
                     count_prev):
    bi = pl.program_id(0)
    x = x_ref[0]
    s = sin_ref[...]
    mu = s[:, 0:1] * (1.0 / count_prev)
    var = s[:, 1:2] * (1.0 / count_prev) - mu * mu
    inv = jax.lax.rsqrt(var + 1e-5)
    xa = jnp.maximum(x - mu, 0.0) * inv
    y = jax.lax.dot_general(w_ref[...], xa, (((1,), (0,)), ((), ())),
                            precision=jax.lax.Precision.DEFAULT,
                            preferred_element_type=jnp.float32) + b_ref[...]
    ps = jnp.sum(y, axis=1, keepdims=True)
    pq = jnp.sum(y * y, axis=1, keepdims=True)
    part = jnp.concatenate([ps, pq], axis=1)

    @pl.when(bi == 0)
    def _():
        st_ref[...] = part

    @pl.when(bi > 0)
    def _():
        st_ref[...] = st_ref[...] + part

    y_ref[0] = y


def _layer(x, w, bias, stats_in, count_prev):
    b, cin, l = x.shape
    cout = w.shape[0]
    outs = (jax.ShapeDtypeStruct((b, cout, l), jnp.float32),
            jax.ShapeDtypeStruct((cout, 2), jnp.float32))
    ospecs = (pl.BlockSpec((1, cout, l), lambda i: (i, 0, 0)),
              pl.BlockSpec((cout, 2), lambda i: (0, 0)))
    if stats_in is None:
        return pl.pallas_call(
            functools.partial(_layer_body, nb=b, count_prev=count_prev),
            grid=(b,),
            in_specs=[
                pl.BlockSpec((1, cin, l), lambda i: (i, 0, 0)),
                pl.BlockSpec((cout, cin), lambda i: (0, 0)),
                pl.BlockSpec((cout, 1), lambda i: (0, 0)),
            ],
            out_specs=ospecs, out_shape=outs,
        )(x, w, bias)
    return pl.pallas_call(
        functools.partial(_layer_norm_body, nb=b, count_prev=count_prev),
        grid=(b,),
        in_specs=[
            pl.BlockSpec((1, cin, l), lambda i: (i, 0, 0)),
            pl.BlockSpec((cin, 2), lambda i: (0, 0)),
            pl.BlockSpec((cout, cin), lambda i: (0, 0)),
            pl.BlockSpec((cout, 1), lambda i: (0, 0)),
        ],
        out_specs=ospecs, out_shape=outs,
    )(x, stats_in, w, bias)


def _pool_body(y_ref, st_ref, out_ref, *, s_, k_, count):
    x = y_ref[0]
    s = st_ref[...]
    mu = s[:, 0:1] * (1.0 / count)
    var = s[:, 1:2] * (1.0 / count) - mu * mu
    inv = jax.lax.rsqrt(var + 1e-5)
    xa = jnp.maximum(x - mu, 0.0) * inv
    acc = xa[:, 0:s_]
    for k in range(1, k_):
        acc = jnp.maximum(acc, xa[:, k * s_:(k + 1) * s_])
    out_ref[0] = acc


def _pool(y, stats, s_, k_, count):
    b, c, _ = y.shape
    return pl.pallas_call(
        functools.partial(_pool_body, s_=s_, k_=k_, count=count),
        grid=(b,),
        in_specs=[
            pl.BlockSpec((1, c, k_ * s_), lambda i: (i, 0, 0)),
            pl.BlockSpec((c, 2), lambda i: (0, 0)),
        ],
        out_specs=pl.BlockSpec((1, c, s_), lambda i: (i, 0, 0)),
        out_shape=jax.ShapeDtypeStruct((b, c, s_), jnp.float32),
    )(y, stats)


def _sa3_body(x_ref, w1, b1, w2, b2, w3, b3, out_ref, *, kk):
    x = x_ref[...]
    cnt = float(x.shape[1])

    def bn_relu(y):
        mu = jnp.sum(y, axis=1, keepdims=True) * (1.0 / cnt)
        var = jnp.sum(y * y, axis=1, keepdims=True) * (1.0 / cnt) - mu * mu
        inv = jax.lax.rsqrt(var + 1e-5)
        return jnp.maximum(y - mu, 0.0) * inv

    def mm(w_ref, b_ref, v):
        return jax.lax.dot_general(
            w_ref[...], v, (((1,), (0,)), ((), ())),
            precision=jax.lax.Precision.DEFAULT,
            preferred_element_type=jnp.float32) + b_ref[...]

    z = bn_relu(mm(w1, b1, x))
    z = bn_relu(mm(w2, b2, z))
    z = bn_relu(mm(w3, b3, z))
    cols = [jnp.max(z[:, b * kk:(b + 1) * kk], axis=1, keepdims=True)
            for b in range(_B)]
    out_ref[...] = jnp.concatenate(cols, axis=1)


def _sa3(x, p3, kk):
    c3 = p3[2][0].shape[0]
    args = [x]
    for w, bias in p3:
        args += [w, bias.reshape(-1, 1)]
    return pl.pallas_call(
        functools.partial(_sa3_body, kk=kk),
        out_shape=jax.ShapeDtypeStruct((c3, _B), jnp.float32),
    )(*args)


def _stage(xyz, pts, p, s_, k_, r2):
    new_xyz = _fps(xyz, s_)
    g = _group(new_xyz, xyz, pts, s_, k_, r2)
    cnt = float(_B * s_ * k_)
    y, st = _layer(g, p[0][0], p[0][1].reshape(-1, 1), None, cnt)
    y, st = _layer(y, p[1][0], p[1][1].reshape(-1, 1), st, cnt)
    y, st = _layer(y, p[2][0], p[2][1].reshape(-1, 1), st, cnt)
    out = _pool(y, st, s_, k_, cnt)
    return new_xyz, out


def kernel(xyz, params):
    l1_xyz, l1_pts = _stage(xyz, None, params[0], 512, 32, 0.2 ** 2)
    l2_xyz, l2_pts = _stage(l1_xyz, l1_pts, params[1], 128, 64, 0.4 ** 2)
    x3 = jnp.concatenate([l2_xyz, l2_pts], axis=1)
    x3 = jnp.transpose(x3, (1, 0, 2)).reshape(x3.shape[1], -1)
    out = _sa3(x3, params[2], l2_pts.shape[2])
    return jnp.transpose(out, (1, 0))


# speedup vs baseline: 1.1787x; 1.1787x over previous
import functools
import jax
import jax.numpy as jnp
from jax.experimental import pallas as pl
from jax.experimental.pallas import tpu as pltpu

_B = 16


def _fps_body(xyz_ref, new_ref, *, npoint, n):
    x = xyz_ref[:, 0, :]
    y = xyz_ref[:, 1, :]
    z = xyz_ref[:, 2, :]
    lane_n = jax.lax.broadcasted_iota(jnp.int32, (_B, n), 1)
    lane_s = jax.lax.broadcasted_iota(jnp.int32, (_B, npoint), 1)

    def body(i, state):
        dist, far, ax, ay, az = state
        oh = (lane_n == far).astype(jnp.float32)
        cx = jnp.sum(oh * x, axis=1, keepdims=True)
        cy = jnp.sum(oh * y, axis=1, keepdims=True)
        cz = jnp.sum(oh * z, axis=1, keepdims=True)
        sel = lane_s == i
        ax = jnp.where(sel, cx, ax)
        ay = jnp.where(sel, cy, ay)
        az = jnp.where(sel, cz, az)
        dx = x - cx
        dy = y - cy
        dz = z - cz
        d = dx * dx + dy * dy + dz * dz
        dist = jnp.minimum(dist, d)
        m = jnp.max(dist, axis=1, keepdims=True)
        far = jnp.min(jnp.where(dist == m, lane_n, n), axis=1, keepdims=True)
        return (dist, far, ax, ay, az)

    init = (
        jnp.full((_B, n), 1e10, jnp.float32),
        jnp.zeros((_B, 1), jnp.int32),
        jnp.zeros((_B, npoint), jnp.float32),
        jnp.zeros((_B, npoint), jnp.float32),
        jnp.zeros((_B, npoint), jnp.float32),
    )
    _, _, ax, ay, az = jax.lax.fori_loop(0, npoint, body, init)
    new_ref[:, 0, :] = ax
    new_ref[:, 1, :] = ay
    new_ref[:, 2, :] = az


def _fps(xyz, npoint):
    b, _, n = xyz.shape
    return pl.pallas_call(
        functools.partial(_fps_body, npoint=npoint, n=n),
        out_shape=jax.ShapeDtypeStruct((b, 3, npoint), jnp.float32),
    )(xyz)


def _group_body(new_ref, xyz_ref, pts_ref, out_ref, *, s_, sc, k_, n, r2, cp):
    j = pl.program_id(1)
    nx = new_ref[0]
    px = xyz_ref[0]
    pts = None if pts_ref is None else pts_ref[0]
    nxT = jnp.transpose(nx)
    ns_col = (nxT[:, 0:1] * nxT[:, 0:1] + nxT[:, 1:2] * nxT[:, 1:2]) \
        + nxT[:, 2:3] * nxT[:, 2:3]
    nd_row = (px[0:1, :] * px[0:1, :] + px[1:2, :] * px[1:2, :]) \
        + px[2:3, :] * px[2:3, :]
    dot = jax.lax.dot_general(nx, px, (((0,), (0,)), ((), ())),
                              precision=jax.lax.Precision.DEFAULT,
                              preferred_element_type=jnp.float32)
    d = (-2.0 * dot + ns_col) + nd_row
    mask_f = (d <= r2).astype(jnp.float32)
    lane = jax.lax.broadcasted_iota(jnp.int32, (sc, n), 1)
    a = mask_f
    sh = 1
    while sh < n:
        rolled = pltpu.roll(a, sh, 1)
        a = a + jnp.where(lane >= sh, rolled, 0.0)
        sh *= 2
    ranksel = jnp.where(mask_f > 0.5, a - mask_f, -1.0)
    rank_t = jnp.transpose(ranksel)
    count_row = jnp.transpose(
        jnp.sum(mask_f, axis=1, keepdims=True))
    stacked = px if pts is None else jnp.concatenate([px, pts], axis=0)
    hi = stacked.astype(jnp.bfloat16)
    r1 = stacked - hi.astype(jnp.float32)
    mid = r1.astype(jnp.bfloat16)
    lo = (r1 - mid.astype(jnp.float32)).astype(jnp.bfloat16)
    g0 = None
    for k in range(k_):
        sel_t = (rank_t == float(k)).astype(jnp.bfloat16)
        dims = (((1,), (0,)), ((), ()))
        g = (jax.lax.dot_general(hi, sel_t, dims,
                                 precision=jax.lax.Precision.DEFAULT,
                                 preferred_element_type=jnp.float32)
             + jax.lax.dot_general(mid, sel_t, dims,
                                   precision=jax.lax.Precision.DEFAULT,
                                   preferred_element_type=jnp.float32)) \
            + jax.lax.dot_general(lo, sel_t, dims,
                                  precision=jax.lax.Precision.DEFAULT,
                                  preferred_element_type=jnp.float32)
        if k == 0:
            empty = (count_row <= 0).astype(jnp.float32)
            g = g + empty * stacked[:, n - 1:n]
            g0 = g
        else:
            pad = (count_row <= k).astype(jnp.float32)
            g = g + pad * g0
        norm = g[0:3] - nx
        rest = g[0:3] if pts is None else g[3:]
        cat = jnp.concatenate([norm, rest], axis=0)
        out_ref[0, :, pl.ds(k * s_ + j * sc, sc)] = cat


def _group(new_xyz, xyz, pts, s_, k_, r2):
    b = xyz.shape[0]
    n = xyz.shape[2]
    cp = 3 if pts is None else pts.shape[1]
    sc = min(s_, 128)
    in_specs = [
        pl.BlockSpec((1, 3, sc), lambda i, j: (i, 0, j)),
        pl.BlockSpec((1, 3, n), lambda i, j: (i, 0, 0)),
    ]
    args = [new_xyz, xyz]
    if pts is None:
        body = functools.partial(
            lambda nr, xr, orf, **kw: _group_body(nr, xr, None, orf, **kw),
            s_=s_, sc=sc, k_=k_, n=n, r2=r2, cp=cp)
    else:
        body = functools.partial(_group_body, s_=s_, sc=sc, k_=k_, n=n,
                                 r2=r2, cp=cp)
        in_specs.append(pl.BlockSpec((1, cp, n), lambda i, j: (i, 0, 0)))
        args.append(pts)
    return pl.pallas_call(
        body,
        grid=(b, s_ // sc),
        in_specs=in_specs,
        out_specs=pl.BlockSpec((1, 3 + cp, k_ * s_), lambda i, j: (i, 0, 0)),
        out_shape=jax.ShapeDtypeStruct((b, 3 + cp, k_ * s_), jnp.float32),
    )(*args)


def _layer_body(x_ref, w_ref, b_ref, y_ref, st_ref, *, nb, count_prev):
    bi = pl.program_id(0)
    x = x_ref[0]
    y = jax.lax.dot_general(w_ref[...], x, (((1,), (0,)), ((), ())),
                            precision=jax.lax.Precision.DEFAULT,
                            preferred_element_type=jnp.float32) + b_ref[...]
    ps = jnp.sum(y, axis=1, keepdims=True)
    pq = jnp.sum(y * y, axis=1, keepdims=True)
    part = jnp.concatenate([ps, pq], axis=1)

    @pl.when(bi == 0)
    def _():
        st_ref[...] = part

    @pl.when(bi > 0)
    def _():
        st_ref[...] = st_ref[...] + part

    y_ref[0] = y


def _layer_norm_body(x_ref, sin_ref, w_ref, b_ref, y_ref, st_ref, *, nb,
                     count_prev):
    bi = pl.program_id(0)
    x = x_ref[0]
    s = sin_ref[...]
    mu = s[:, 0:1] * (1.0 / count_prev)
    var = s[:, 1:2] * (1.0 / count_prev) - mu * mu
    inv = jax.lax.rsqrt(var + 1e-5)
    xa = jnp.maximum(x - mu, 0.0) * inv
    y = jax.lax.dot_general(w_ref[...], xa, (((1,), (0,)), ((), ())),
                            precision=jax.lax.Precision.DEFAULT,
                            preferred_element_type=jnp.float32) + b_ref[...]
    ps = jnp.sum(y, axis=1, keepdims=True)
    pq = jnp.sum(y * y, axis=1, keepdims=True)
    part = jnp.concatenate([ps, pq], axis=1)

    @pl.when(bi == 0)
    def _():
        st_ref[...] = part

    @pl.when(bi > 0)
    def _():
        st_ref[...] = st_ref[...] + part

    y_ref[0] = y


def _layer(x, w, bias, stats_in, count_prev):
    b, cin, l = x.shape
    cout = w.shape[0]
    outs = (jax.ShapeDtypeStruct((b, cout, l), jnp.float32),
            jax.ShapeDtypeStruct((cout, 2), jnp.float32))
    ospecs = (pl.BlockSpec((1, cout, l), lambda i: (i, 0, 0)),
              pl.BlockSpec((cout, 2), lambda i: (0, 0)))
    if stats_in is None:
        return pl.pallas_call(
            functools.partial(_layer_body, nb=b, count_prev=count_prev),
            grid=(b,),
            in_specs=[
                pl.BlockSpec((1, cin, l), lambda i: (i, 0, 0)),
                pl.BlockSpec((cout, cin), lambda i: (0, 0)),
                pl.BlockSpec((cout, 1), lambda i: (0, 0)),
            ],
            out_specs=ospecs, out_shape=outs,
        )(x, w, bias)
    return pl.pallas_call(
        functools.partial(_layer_norm_body, nb=b, count_prev=count_prev),
        grid=(b,),
        in_specs=[
            pl.BlockSpec((1, cin, l), lambda i: (i, 0, 0)),
            pl.BlockSpec((cin, 2), lambda i: (0, 0)),
            pl.BlockSpec((cout, cin), lambda i: (0, 0)),
            pl.BlockSpec((cout, 1), lambda i: (0, 0)),
        ],
        out_specs=ospecs, out_shape=outs,
    )(x, stats_in, w, bias)


def _pool_body(y_ref, st_ref, out_ref, *, s_, k_, count):
    x = y_ref[0]
    s = st_ref[...]
    mu = s[:, 0:1] * (1.0 / count)
    var = s[:, 1:2] * (1.0 / count) - mu * mu
    inv = jax.lax.rsqrt(var + 1e-5)
    xa = jnp.maximum(x - mu, 0.0) * inv
    acc = xa[:, 0:s_]
    for k in range(1, k_):
        acc = jnp.maximum(acc, xa[:, k * s_:(k + 1) * s_])
    out_ref[0] = acc


def _pool(y, stats, s_, k_, count):
    b, c, _ = y.shape
    return pl.pallas_call(
        functools.partial(_pool_body, s_=s_, k_=k_, count=count),
        grid=(b,),
        in_specs=[
            pl.BlockSpec((1, c, k_ * s_), lambda i: (i, 0, 0)),
            pl.BlockSpec((c, 2), lambda i: (0, 0)),
        ],
        out_specs=pl.BlockSpec((1, c, s_), lambda i: (i, 0, 0)),
        out_shape=jax.ShapeDtypeStruct((b, c, s_), jnp.float32),
    )(y, stats)


def _sa3_body(x_ref, w1, b1, w2, b2, w3, b3, out_ref, *, kk):
    x = x_ref[...]
    cnt = float(x.shape[1])

    def bn_relu(y):
        mu = jnp.sum(y, axis=1, keepdims=True) * (1.0 / cnt)
        var = jnp.sum(y * y, axis=1, keepdims=True) * (1.0 / cnt) - mu * mu
        inv = jax.lax.rsqrt(var + 1e-5)
        return jnp.maximum(y - mu, 0.0) * inv

    def mm(w_ref, b_ref, v):
        return jax.lax.dot_general(
            w_ref[...], v, (((1,), (0,)), ((), ())),
            precision=jax.lax.Precision.DEFAULT,
            preferred_element_type=jnp.float32) + b_ref[...]

    z = bn_relu(mm(w1, b1, x))
    z = bn_relu(mm(w2, b2, z))
    z = bn_relu(mm(w3, b3, z))
    cols = [jnp.max(z[:, b * kk:(b + 1) * kk], axis=1, keepdims=True)
            for b in range(_B)]
    out_ref[...] = jnp.concatenate(cols, axis=1)


def _sa3(x, p3, kk):
    c3 = p3[2][0].shape[0]
    args = [x]
    for w, bias in p3:
        args += [w, bias.reshape(-1, 1)]
    return pl.pallas_call(
        functools.partial(_sa3_body, kk=kk),
        out_shape=jax.ShapeDtypeStruct((c3, _B), jnp.float32),
    )(*args)


def _stage(xyz, pts, p, s_, k_, r2):
    new_xyz = _fps(xyz, s_)
    g = _group(new_xyz, xyz, pts, s_, k_, r2)
    cnt = float(_B * s_ * k_)
    y, st = _layer(g, p[0][0], p[0][1].reshape(-1, 1), None, cnt)
    y, st = _layer(y, p[1][0], p[1][1].reshape(-1, 1), st, cnt)
    y, st = _layer(y, p[2][0], p[2][1].reshape(-1, 1), st, cnt)
    out = _pool(y, st, s_, k_, cnt)
    return new_xyz, out


def kernel(xyz, params):
    l1_xyz, l1_pts = _stage(xyz, None, params[0], 512, 32, 0.2 ** 2)
    l2_xyz, l2_pts = _stage(l1_xyz, l1_pts, params[1], 128, 64, 0.4 ** 2)
    x3 = jnp.concatenate([l2_xyz, l2_pts], axis=1)
    x3 = jnp.transpose(x3, (1, 0, 2)).reshape(x3.shape[1], -1)
    out = _sa3(x3, params[2], l2_pts.shape[2])
    return jnp.transpose(out, (1, 0))
